# baseline (speedup 1.0000x reference)
import functools

import jax
import jax.numpy as jnp
from jax import lax
from jax.experimental import pallas as pl
from jax.experimental.pallas import tpu as pltpu
from jax.experimental.pallas import tpu_sc as plsc

N_NODES = 50000
N_EDGES = 800000
NUM_IDS = 2048
IN_CH = 11
HID = 32
LAT = 32
HEADS = 4
EMB = 8

NPAD = 50176
EG = 851968
EE = 802816
IDPAD = 53248

NC, NS = 2, 16
NW = NC * NS
K = 128

BN = 512
BE = 2048

_f32 = jnp.float32
_i32 = jnp.int32


@functools.cache
def _make_gather(nrows, d, epad):
    del nrows
    ew = epad // NW
    rounds = ew // K
    mesh = plsc.VectorSubcoreMesh(core_axis_name="c", subcore_axis_name="s")

    def body(table_hbm, idx_hbm, out_hbm, idx_v, rows_v, sem):
        wid = lax.axis_index("s") * NC + lax.axis_index("c")

        def step(r, carry):
            base = wid * ew + r * K
            pltpu.sync_copy(idx_hbm.at[pl.ds(base, K)], idx_v)
            pltpu.async_copy(table_hbm.at[idx_v], rows_v, sem).wait()
            pltpu.sync_copy(rows_v, out_hbm.at[pl.ds(base, K)])
            return carry

        lax.fori_loop(0, rounds, step, 0)

    return functools.partial(
        pl.kernel,
        out_type=jax.ShapeDtypeStruct((epad, d), _f32),
        mesh=mesh,
        compiler_params=pltpu.CompilerParams(use_tc_tiling_on_sc=False),
        scratch_types=[
            pltpu.VMEM((K,), _i32),
            pltpu.VMEM((K, d), _f32),
            pltpu.SemaphoreType.DMA,
        ],
    )(body)


@functools.cache
def _make_scatter_add(d, epad):
    ew = epad // NW
    rounds = ew // K
    rpt = NPAD // NS
    mesh = plsc.VectorSubcoreMesh(core_axis_name="c", subcore_axis_name="s")

    def body(vals_hbm, idx_hbm, zeros_hbm, out_hbm, idx_v, val_v, shared):
        cid = lax.axis_index("c")
        sid = lax.axis_index("s")

        @pl.when(sid == 0)
        def _init():
            pltpu.sync_copy(zeros_hbm, shared)

        plsc.subcore_barrier()
        wid = sid * NC + cid

        def step(r, carry):
            base = wid * ew + r * K
            pltpu.sync_copy(idx_hbm.at[pl.ds(base, K)], idx_v)
            pltpu.sync_copy(vals_hbm.at[pl.ds(base, K)], val_v)
            pltpu.sync_copy(val_v, shared.at[idx_v], add=True)
            return carry

        lax.fori_loop(0, rounds, step, 0)
        plsc.subcore_barrier()
        pltpu.sync_copy(
            shared.at[pl.ds(sid * rpt, rpt)],
            out_hbm.at[pl.ds(cid * NPAD + sid * rpt, rpt)],
        )

    return functools.partial(
        pl.kernel,
        out_type=jax.ShapeDtypeStruct((NC * NPAD, d), _f32),
        mesh=mesh,
        compiler_params=pltpu.CompilerParams(use_tc_tiling_on_sc=False),
        scratch_types=[
            pltpu.VMEM((K,), _i32),
            pltpu.VMEM((K, d), _f32),
            pltpu.VMEM_SHARED((NPAD, d), _f32),
        ],
    )(body)


def _leaky(v):
    return jnp.where(v > 0, v, 0.2 * v)


@functools.cache
def _make_stage_a(dinp, cstore, heads):
    hdp = cstore // heads
    grid = NPAD // BN

    def body(x_ref, w_ref, asrc_ref, adst_ref, p_ref, q_ref):
        h = jnp.dot(x_ref[...], w_ref[...], preferred_element_type=_f32)
        p_ref[...] = h
        cols = []
        for hh in range(heads):
            blk = h[:, hh * hdp:(hh + 1) * hdp]
            cols.append(jnp.sum(blk * asrc_ref[hh:hh + 1, :], axis=1, keepdims=True))
        for hh in range(heads):
            blk = h[:, hh * hdp:(hh + 1) * hdp]
            cols.append(jnp.sum(blk * adst_ref[hh:hh + 1, :], axis=1, keepdims=True))
        cols.append(jnp.zeros((BN, 16 - 2 * heads), _f32))
        q_ref[...] = jnp.concatenate(cols, axis=1)

    return pl.pallas_call(
        body,
        grid=(grid,),
        in_specs=[
            pl.BlockSpec((BN, dinp), lambda i: (i, 0)),
            pl.BlockSpec((dinp, cstore), lambda i: (0, 0)),
            pl.BlockSpec((heads, hdp), lambda i: (0, 0)),
            pl.BlockSpec((heads, hdp), lambda i: (0, 0)),
        ],
        out_specs=[
            pl.BlockSpec((BN, cstore), lambda i: (i, 0)),
            pl.BlockSpec((BN, 16), lambda i: (i, 0)),
        ],
        out_shape=[
            jax.ShapeDtypeStruct((NPAD, cstore), _f32),
            jax.ShapeDtypeStruct((NPAD, 16), _f32),
        ],
    )


@functools.cache
def _make_stage_c(heads, epad):
    c = heads * 32
    grid = epad // BE

    def body(gh_ref, qs_ref, qd_ref, v1_ref, v2_ref):
        exs = []
        for hh in range(heads):
            e = qs_ref[:, hh:hh + 1] + qd_ref[:, heads + hh:heads + hh + 1]
            ex = jnp.exp(_leaky(e))
            exs.append(ex)
            v1_ref[hh, :, :] = gh_ref[:, hh * 32:(hh + 1) * 32] * ex
        exs.append(jnp.zeros((BE, 16 - heads), _f32))
        v2_ref[...] = jnp.concatenate(exs, axis=1)

    return pl.pallas_call(
        body,
        grid=(grid,),
        in_specs=[
            pl.BlockSpec((BE, c), lambda i: (i, 0)),
            pl.BlockSpec((BE, 16), lambda i: (i, 0)),
            pl.BlockSpec((BE, 16), lambda i: (i, 0)),
        ],
        out_specs=[
            pl.BlockSpec((heads, BE, 32), lambda i: (0, i, 0)),
            pl.BlockSpec((BE, 16), lambda i: (i, 0)),
        ],
        out_shape=[
            jax.ShapeDtypeStruct((heads, epad, 32), _f32),
            jax.ShapeDtypeStruct((epad, 16), _f32),
        ],
    )


@functools.cache
def _make_stage_c16(epad):
    grid = epad // BE

    def body(gh_ref, qs_ref, qd_ref, v_ref):
        e = qs_ref[:, 0:1] + qd_ref[:, 1:2]
        ex = jnp.exp(_leaky(e))
        v_ref[...] = jnp.concatenate(
            [gh_ref[:, :10] * ex, ex, jnp.zeros((BE, 5), _f32)], axis=1)

    return pl.pallas_call(
        body,
        grid=(grid,),
        in_specs=[
            pl.BlockSpec((BE, 16), lambda i: (i, 0)),
            pl.BlockSpec((BE, 16), lambda i: (i, 0)),
            pl.BlockSpec((BE, 16), lambda i: (i, 0)),
        ],
        out_specs=pl.BlockSpec((BE, 16), lambda i: (i, 0)),
        out_shape=jax.ShapeDtypeStruct((epad, 16), _f32),
    )


@functools.cache
def _make_stage_e(heads, bn_relu):
    c = heads * 32
    grid = NPAD // BN

    def body(*refs):
        acc_refs = refs[:heads]
        den_ref, bias_ref, g_ref, b_ref, out_ref = refs[heads:]
        planes = []
        for hh in range(heads):
            num = acc_refs[hh][0] + acc_refs[hh][1]
            den = den_ref[0, :, hh:hh + 1] + den_ref[1, :, hh:hh + 1] + 1e-16
            planes.append(num / den)
        o = jnp.concatenate(planes, axis=1) + bias_ref[...]
        if bn_relu:
            o = jnp.maximum(o * g_ref[...] + b_ref[...], 0.0)
        out_ref[...] = o

    return pl.pallas_call(
        body,
        grid=(grid,),
        in_specs=(
            [pl.BlockSpec((2, BN, 32), lambda i: (0, i, 0)) for _ in range(heads)]
            + [
                pl.BlockSpec((2, BN, 16), lambda i: (0, i, 0)),
                pl.BlockSpec((1, c), lambda i: (0, 0)),
                pl.BlockSpec((1, c), lambda i: (0, 0)),
                pl.BlockSpec((1, c), lambda i: (0, 0)),
            ]
        ),
        out_specs=pl.BlockSpec((BN, c), lambda i: (i, 0)),
        out_shape=jax.ShapeDtypeStruct((NPAD, c), _f32),
    )


@functools.cache
def _make_stage_e16():
    grid = NPAD // BN

    def body(acc_ref, bias_ref, out_ref):
        num = acc_ref[0] + acc_ref[1]
        val = num[:, :10] / (num[:, 10:11] + 1e-16)
        out_ref[...] = jnp.concatenate(
            [val, jnp.zeros((BN, 6), _f32)], axis=1) + bias_ref[...]

    return pl.pallas_call(
        body,
        grid=(grid,),
        in_specs=[
            pl.BlockSpec((2, BN, 16), lambda i: (0, i, 0)),
            pl.BlockSpec((1, 16), lambda i: (0, 0)),
        ],
        out_specs=pl.BlockSpec((BN, 16), lambda i: (i, 0)),
        out_shape=jax.ShapeDtypeStruct((NPAD, 16), _f32),
    )


@functools.cache
def _make_latent():
    grid = NPAD // BN

    def body(x2_ref, x3_ref, zmw_ref, zmb_ref, zlw_ref, zlb_ref, eps_ref,
             z_ref, kl_ref, acc):
        i = pl.program_id(0)
        h = x2_ref[...] + x3_ref[...]
        mu = jnp.dot(h, zmw_ref[...], preferred_element_type=_f32) + zmb_ref[...]
        lv = jnp.dot(h, zlw_ref[...], preferred_element_type=_f32) + zlb_ref[...]
        z_ref[...] = mu + eps_ref[...] * jnp.exp(0.5 * lv)
        rows = lax.broadcasted_iota(_i32, (BN, 1), 0) + i * BN
        term = jnp.where(rows < N_NODES, lv - mu * mu - jnp.exp(lv), 0.0)

        @pl.when(i == 0)
        def _init():
            acc[0, 0] = 0.0

        acc[0, 0] += jnp.sum(term)

        @pl.when(i == grid - 1)
        def _fin():
            val = -0.5 * (1.0 + acc[0, 0] / (N_NODES * LAT))
            kl_ref[...] = jnp.zeros((1, 1), _f32) + val

    return pl.pallas_call(
        body,
        grid=(grid,),
        in_specs=[
            pl.BlockSpec((BN, LAT), lambda i: (i, 0)),
            pl.BlockSpec((BN, LAT), lambda i: (i, 0)),
            pl.BlockSpec((LAT, LAT), lambda i: (0, 0)),
            pl.BlockSpec((1, LAT), lambda i: (0, 0)),
            pl.BlockSpec((LAT, LAT), lambda i: (0, 0)),
            pl.BlockSpec((1, LAT), lambda i: (0, 0)),
            pl.BlockSpec((BN, LAT), lambda i: (i, 0)),
        ],
        out_specs=[
            pl.BlockSpec((BN, LAT), lambda i: (i, 0)),
            pl.BlockSpec((1, 1), lambda i: (0, 0)),
        ],
        out_shape=[
            jax.ShapeDtypeStruct((NPAD, LAT), _f32),
            jax.ShapeDtypeStruct((1, 1), _f32),
        ],
        scratch_shapes=[pltpu.SMEM((1, 1), _f32)],
    )


@functools.cache
def _make_classifier():
    bn = 400
    grid = N_NODES // bn

    def body(z_ref, w_ref, b_ref, out_ref):
        out_ref[...] = (
            jnp.dot(z_ref[...], w_ref[...], preferred_element_type=_f32)
            + b_ref[...])

    return pl.pallas_call(
        body,
        grid=(grid,),
        in_specs=[
            pl.BlockSpec((bn, LAT), lambda i: (i, 0)),
            pl.BlockSpec((LAT, NUM_IDS), lambda i: (0, 0)),
            pl.BlockSpec((1, NUM_IDS), lambda i: (0, 0)),
        ],
        out_specs=pl.BlockSpec((bn, NUM_IDS), lambda i: (i, 0)),
        out_shape=jax.ShapeDtypeStruct((N_NODES, NUM_IDS), _f32),
    )


@functools.cache
def _make_edge_mlp():
    grid = EE // BE

    def body(zs_ref, zd_ref, w1_ref, b1_ref, g1_ref, bb1_ref,
             w2_ref, b2_ref, g2_ref, bb2_ref, w3_ref, b3_ref, out_ref):
        zs = zs_ref[...]
        zd = zd_ref[...]
        ef = jnp.concatenate([zs, zd, zs * zd, jnp.abs(zs - zd)], axis=1)
        e1 = jnp.dot(ef, w1_ref[...], preferred_element_type=_f32) + b1_ref[...]
        e1 = jnp.maximum(e1 * g1_ref[...] + bb1_ref[...], 0.0)
        e2 = jnp.dot(e1, w2_ref[...], preferred_element_type=_f32) + b2_ref[...]
        e2 = jnp.maximum(e2 * g2_ref[...] + bb2_ref[...], 0.0)
        o = jnp.dot(e2, w3_ref[...], preferred_element_type=_f32) + b3_ref[...]
        out_ref[...] = jax.nn.sigmoid(o)

    return pl.pallas_call(
        body,
        grid=(grid,),
        in_specs=[
            pl.BlockSpec((BE, LAT), lambda i: (i, 0)),
            pl.BlockSpec((BE, LAT), lambda i: (i, 0)),
            pl.BlockSpec((4 * LAT, 128), lambda i: (0, 0)),
            pl.BlockSpec((1, 128), lambda i: (0, 0)),
            pl.BlockSpec((1, 128), lambda i: (0, 0)),
            pl.BlockSpec((1, 128), lambda i: (0, 0)),
            pl.BlockSpec((128, 64), lambda i: (0, 0)),
            pl.BlockSpec((1, 64), lambda i: (0, 0)),
            pl.BlockSpec((1, 64), lambda i: (0, 0)),
            pl.BlockSpec((1, 64), lambda i: (0, 0)),
            pl.BlockSpec((64, 8), lambda i: (0, 0)),
            pl.BlockSpec((1, 8), lambda i: (0, 0)),
        ],
        out_specs=pl.BlockSpec((BE, 8), lambda i: (i, 0)),
        out_shape=jax.ShapeDtypeStruct((EE, 8), _f32),
    )


_BN_SCALE = 1.0 / (1.0 + 1e-5) ** 0.5


def _row(v):
    return v.reshape(1, -1)


def _pad_rows(a, rows, val=0.0):
    return jnp.pad(a, ((0, rows - a.shape[0]), (0, 0)), constant_values=val)


def _gat_layer(xp, srcp, dstp, W, asrc, adst, bias, heads, zeros32, zeros16,
               g=None, b=None, cstore=None):
    dinp = xp.shape[1]
    if cstore is None:
        cstore = heads * 32
    hdp = cstore // heads
    asrc_p = jnp.pad(asrc, ((0, 0), (0, hdp - asrc.shape[1])))
    adst_p = jnp.pad(adst, ((0, 0), (0, hdp - adst.shape[1])))
    Wp = jnp.pad(W, ((0, dinp - W.shape[0]), (0, cstore - W.shape[1])))

    P, Q = _make_stage_a(dinp, cstore, heads)(xp, Wp, asrc_p, adst_p)
    Gh = _make_gather(NPAD, cstore, EG)(P, srcp)
    Qs = _make_gather(NPAD, 16, EG)(Q, srcp)
    Qd = _make_gather(NPAD, 16, EG)(Q, dstp)
    if cstore == 16:
        V = _make_stage_c16(EG)(Gh, Qs, Qd)
        acc = _make_scatter_add(16, EG)(V, dstp, zeros16).reshape(2, NPAD, 16)
        bias_p = _row(jnp.pad(bias, (0, 16 - bias.shape[0])))
        return _make_stage_e16()(acc, bias_p)
    V1, V2 = _make_stage_c(heads, EG)(Gh, Qs, Qd)
    accs = [
        _make_scatter_add(32, EG)(V1[hh], dstp, zeros32).reshape(2, NPAD, 32)
        for hh in range(heads)
    ]
    den = _make_scatter_add(16, EG)(V2, dstp, zeros16).reshape(2, NPAD, 16)
    bn_relu = g is not None
    if bn_relu:
        g_p = _row(g) * _BN_SCALE
        b_p = _row(b)
    else:
        g_p = _row(jnp.ones((cstore,), _f32))
        b_p = _row(jnp.zeros((cstore,), _f32))
    return _make_stage_e(heads, bn_relu)(*accs, den, _row(bias), g_p, b_p)


@jax.jit
def kernel(x, edge_index, params):
    p = params

    ids = x[:, 0].astype(_i32)
    ids_p = jnp.pad(ids, (0, IDPAD - N_NODES))
    emb_p = jnp.pad(p['emb'], ((0, 0), (0, 16 - EMB)))
    id_emb = _make_gather(NUM_IDS, 16, IDPAD)(emb_p, ids_p)[:NPAD, :EMB]
    feats = _pad_rows(x[:, 1:], NPAD)
    xin = jnp.concatenate(
        [id_emb, feats, jnp.zeros((NPAD, 32 - EMB - (IN_CH - 1)), _f32)], axis=1)

    loops = jnp.arange(N_NODES, dtype=_i32)
    srcg = jnp.pad(jnp.concatenate([edge_index[0], loops]),
                   (0, EG - N_EDGES - N_NODES), constant_values=N_NODES)
    dstg = jnp.pad(jnp.concatenate([edge_index[1], loops]),
                   (0, EG - N_EDGES - N_NODES), constant_values=N_NODES)
    srce = jnp.pad(edge_index[0], (0, EE - N_EDGES), constant_values=N_NODES)
    dste = jnp.pad(edge_index[1], (0, EE - N_EDGES), constant_values=N_NODES)
    zeros32 = jnp.zeros((NPAD, 32), _f32)
    zeros16 = jnp.zeros((NPAD, 16), _f32)
    eps = _pad_rows(
        jax.random.normal(jax.random.key(42), (N_NODES, LAT), dtype=_f32), NPAD)

    x1 = _gat_layer(xin, srcg, dstg, p['enc1_W'], p['enc1_asrc'],
                    p['enc1_adst'], p['enc1_b'], HEADS, zeros32, zeros16,
                    g=p['bn1_g'], b=p['bn1_b'])
    x2 = _gat_layer(x1, srcg, dstg, p['enc2_W'], p['enc2_asrc'],
                    p['enc2_adst'], p['enc2_b'], 1, zeros32, zeros16,
                    g=p['bn2_g'], b=p['bn2_b'])
    x3 = _gat_layer(x2, srcg, dstg, p['enc3_W'], p['enc3_asrc'],
                    p['enc3_adst'], p['enc3_b'], 1, zeros32, zeros16,
                    g=p['bn3_g'], b=p['bn3_b'])

    z, kl = _make_latent()(x2, x3, p['zm_W'], _row(p['zm_b']),
                           p['zl_W'], _row(p['zl_b']), eps)

    d1 = _gat_layer(z, srcg, dstg, p['dec1_W'], p['dec1_asrc'],
                    p['dec1_adst'], p['dec1_b'], HEADS, zeros32, zeros16,
                    g=p['dbn1_g'], b=p['dbn1_b'])
    d2 = _gat_layer(d1, srcg, dstg, p['dec2_W'], p['dec2_asrc'],
                    p['dec2_adst'], p['dec2_b'], 1, zeros32, zeros16,
                    g=p['dbn2_g'], b=p['dbn2_b'])
    nr = _gat_layer(d2, srcg, dstg, p['dec3_W'], p['dec3_asrc'],
                    p['dec3_adst'], p['dec3_b'], 1, zeros32, zeros16,
                    cstore=16)
    node_recon = nr[:N_NODES, :IN_CH - 1]

    canid_logits = _make_classifier()(z, p['cls_W'], _row(p['cls_b']))

    zs = _make_gather(NPAD, 32, EE)(z, srce)
    zd = _make_gather(NPAD, 32, EE)(z, dste)
    w3 = jnp.pad(p['ed3_W'], ((0, 0), (0, 7)))
    b3 = _row(jnp.pad(p['ed3_b'], (0, 7)))
    ep = _make_edge_mlp()(
        zs, zd,
        p['ed1_W'], _row(p['ed1_b']),
        _row(p['edbn1_g']) * _BN_SCALE, _row(p['edbn1_b']),
        p['ed2_W'], _row(p['ed2_b']),
        _row(p['edbn2_g']) * _BN_SCALE, _row(p['edbn2_b']),
        w3, b3)
    edge_prob = ep[:N_EDGES, :1]

    return (node_recon, canid_logits, edge_prob, kl[0, 0])


# speedup vs baseline: 1.1873x; 1.1873x over previous
import functools

import jax
import jax.numpy as jnp
from jax import lax
from jax.experimental import pallas as pl
from jax.experimental.pallas import tpu as pltpu
from jax.experimental.pallas import tpu_sc as plsc

N_NODES = 50000
N_EDGES = 800000
NUM_IDS = 2048
IN_CH = 11
HID = 32
LAT = 32
HEADS = 4
EMB = 8

NPAD = 50176
EG = 851968
EE = 802816
IDPAD = 53248

NC, NS = 2, 16
NW = NC * NS
K = 128

BN = 512
BE = 2048

_f32 = jnp.float32
_i32 = jnp.int32


def _super_round(d, ew, cap=262144):
    s = min(1024, cap // (4 * d))
    while ew % s:
        s //= 2
    return s


@functools.cache
def _make_gather(nrows, d, epad):
    del nrows
    ew = epad // NW
    S = _super_round(d, ew)
    nk = S // K
    srounds = ew // S
    mesh = plsc.VectorSubcoreMesh(core_axis_name="c", subcore_axis_name="s")

    def body(table_hbm, idx_hbm, out_hbm, idx_v, rows_v, sem):
        wid = lax.axis_index("s") * NC + lax.axis_index("c")

        def step(r, carry):
            base = wid * ew + r * S
            pltpu.sync_copy(idx_hbm.at[pl.ds(base // K, nk)], idx_v)
            cps = [
                pltpu.async_copy(
                    table_hbm.at[idx_v.at[j]],
                    rows_v.at[pl.ds(j * K, K)], sem)
                for j in range(nk)
            ]
            for cp in cps:
                cp.wait()
            pltpu.sync_copy(rows_v, out_hbm.at[pl.ds(base, S)])
            return carry

        lax.fori_loop(0, srounds, step, 0)

    return functools.partial(
        pl.kernel,
        out_type=jax.ShapeDtypeStruct((epad, d), _f32),
        mesh=mesh,
        compiler_params=pltpu.CompilerParams(use_tc_tiling_on_sc=False),
        scratch_types=[
            pltpu.VMEM((S // K, K), _i32),
            pltpu.VMEM((S, d), _f32),
            pltpu.SemaphoreType.DMA,
        ],
    )(body)


@functools.cache
def _make_scatter_add(d, epad):
    ew = epad // NW
    S = _super_round(d, ew, cap=65536)
    nk = S // K
    srounds = ew // S
    rpt = NPAD // NS
    mesh = plsc.VectorSubcoreMesh(core_axis_name="c", subcore_axis_name="s")

    def body(vals_hbm, idx_hbm, zeros_hbm, out_hbm, idx_v, val_v, shared, lsem, ssem):
        cid = lax.axis_index("c")
        sid = lax.axis_index("s")

        @pl.when(sid == 0)
        def _init():
            pltpu.sync_copy(zeros_hbm, shared)

        plsc.subcore_barrier()
        wid = sid * NC + cid

        def step(r, carry):
            base = wid * ew + r * S
            cp1 = pltpu.async_copy(idx_hbm.at[pl.ds(base // K, nk)], idx_v, lsem)
            cp2 = pltpu.async_copy(vals_hbm.at[pl.ds(base, S)], val_v, lsem)
            cp1.wait()
            cp2.wait()
            cps = [
                pltpu.async_copy(
                    val_v.at[pl.ds(j * K, K)],
                    shared.at[idx_v.at[j]], ssem, add=True)
                for j in range(nk)
            ]
            for cp in cps:
                cp.wait()
            return carry

        lax.fori_loop(0, srounds, step, 0)
        plsc.subcore_barrier()
        pltpu.sync_copy(
            shared.at[pl.ds(sid * rpt, rpt)],
            out_hbm.at[pl.ds(cid * NPAD + sid * rpt, rpt)],
        )

    return functools.partial(
        pl.kernel,
        out_type=jax.ShapeDtypeStruct((NC * NPAD, d), _f32),
        mesh=mesh,
        compiler_params=pltpu.CompilerParams(use_tc_tiling_on_sc=False),
        scratch_types=[
            pltpu.VMEM((S // K, K), _i32),
            pltpu.VMEM((S, d), _f32),
            pltpu.VMEM_SHARED((NPAD, d), _f32),
            pltpu.SemaphoreType.DMA,
            pltpu.SemaphoreType.DMA,
        ],
    )(body)


def _leaky(v):
    return jnp.where(v > 0, v, 0.2 * v)


@functools.cache
def _make_stage_a(dinp, cstore, heads):
    hdp = cstore // heads
    grid = NPAD // BN

    def body(x_ref, w_ref, asrc_ref, adst_ref, p_ref, q_ref):
        h = jnp.dot(x_ref[...], w_ref[...], preferred_element_type=_f32)
        p_ref[...] = h
        cols = []
        for hh in range(heads):
            blk = h[:, hh * hdp:(hh + 1) * hdp]
            cols.append(jnp.sum(blk * asrc_ref[hh:hh + 1, :], axis=1, keepdims=True))
        for hh in range(heads):
            blk = h[:, hh * hdp:(hh + 1) * hdp]
            cols.append(jnp.sum(blk * adst_ref[hh:hh + 1, :], axis=1, keepdims=True))
        cols.append(jnp.zeros((BN, 16 - 2 * heads), _f32))
        q_ref[...] = jnp.concatenate(cols, axis=1)

    return pl.pallas_call(
        body,
        grid=(grid,),
        in_specs=[
            pl.BlockSpec((BN, dinp), lambda i: (i, 0)),
            pl.BlockSpec((dinp, cstore), lambda i: (0, 0)),
            pl.BlockSpec((heads, hdp), lambda i: (0, 0)),
            pl.BlockSpec((heads, hdp), lambda i: (0, 0)),
        ],
        out_specs=[
            pl.BlockSpec((BN, cstore), lambda i: (i, 0)),
            pl.BlockSpec((BN, 16), lambda i: (i, 0)),
        ],
        out_shape=[
            jax.ShapeDtypeStruct((NPAD, cstore), _f32),
            jax.ShapeDtypeStruct((NPAD, 16), _f32),
        ],
    )


@functools.cache
def _make_stage_c(heads, epad):
    c = heads * 32
    grid = epad // BE

    def body(gh_ref, qs_ref, qd_ref, v1_ref, v2_ref):
        exs = []
        for hh in range(heads):
            e = qs_ref[:, hh:hh + 1] + qd_ref[:, heads + hh:heads + hh + 1]
            ex = jnp.exp(_leaky(e))
            exs.append(ex)
            v1_ref[hh, :, :] = gh_ref[:, hh * 32:(hh + 1) * 32] * ex
        exs.append(jnp.zeros((BE, 16 - heads), _f32))
        v2_ref[...] = jnp.concatenate(exs, axis=1)

    return pl.pallas_call(
        body,
        grid=(grid,),
        in_specs=[
            pl.BlockSpec((BE, c), lambda i: (i, 0)),
            pl.BlockSpec((BE, 16), lambda i: (i, 0)),
            pl.BlockSpec((BE, 16), lambda i: (i, 0)),
        ],
        out_specs=[
            pl.BlockSpec((heads, BE, 32), lambda i: (0, i, 0)),
            pl.BlockSpec((BE, 16), lambda i: (i, 0)),
        ],
        out_shape=[
            jax.ShapeDtypeStruct((heads, epad, 32), _f32),
            jax.ShapeDtypeStruct((epad, 16), _f32),
        ],
    )


@functools.cache
def _make_stage_c16(epad):
    grid = epad // BE

    def body(gh_ref, qs_ref, qd_ref, v_ref):
        e = qs_ref[:, 0:1] + qd_ref[:, 1:2]
        ex = jnp.exp(_leaky(e))
        v_ref[...] = jnp.concatenate(
            [gh_ref[:, :10] * ex, ex, jnp.zeros((BE, 5), _f32)], axis=1)

    return pl.pallas_call(
        body,
        grid=(grid,),
        in_specs=[
            pl.BlockSpec((BE, 16), lambda i: (i, 0)),
            pl.BlockSpec((BE, 16), lambda i: (i, 0)),
            pl.BlockSpec((BE, 16), lambda i: (i, 0)),
        ],
        out_specs=pl.BlockSpec((BE, 16), lambda i: (i, 0)),
        out_shape=jax.ShapeDtypeStruct((epad, 16), _f32),
    )


@functools.cache
def _make_stage_e(heads, bn_relu):
    c = heads * 32
    grid = NPAD // BN

    def body(*refs):
        acc_refs = refs[:heads]
        den_ref, bias_ref, g_ref, b_ref, out_ref = refs[heads:]
        planes = []
        for hh in range(heads):
            num = acc_refs[hh][0] + acc_refs[hh][1]
            den = den_ref[0, :, hh:hh + 1] + den_ref[1, :, hh:hh + 1] + 1e-16
            planes.append(num / den)
        o = jnp.concatenate(planes, axis=1) + bias_ref[...]
        if bn_relu:
            o = jnp.maximum(o * g_ref[...] + b_ref[...], 0.0)
        out_ref[...] = o

    return pl.pallas_call(
        body,
        grid=(grid,),
        in_specs=(
            [pl.BlockSpec((2, BN, 32), lambda i: (0, i, 0)) for _ in range(heads)]
            + [
                pl.BlockSpec((2, BN, 16), lambda i: (0, i, 0)),
                pl.BlockSpec((1, c), lambda i: (0, 0)),
                pl.BlockSpec((1, c), lambda i: (0, 0)),
                pl.BlockSpec((1, c), lambda i: (0, 0)),
            ]
        ),
        out_specs=pl.BlockSpec((BN, c), lambda i: (i, 0)),
        out_shape=jax.ShapeDtypeStruct((NPAD, c), _f32),
    )


@functools.cache
def _make_stage_e16():
    grid = NPAD // BN

    def body(acc_ref, bias_ref, out_ref):
        num = acc_ref[0] + acc_ref[1]
        val = num[:, :10] / (num[:, 10:11] + 1e-16)
        out_ref[...] = jnp.concatenate(
            [val, jnp.zeros((BN, 6), _f32)], axis=1) + bias_ref[...]

    return pl.pallas_call(
        body,
        grid=(grid,),
        in_specs=[
            pl.BlockSpec((2, BN, 16), lambda i: (0, i, 0)),
            pl.BlockSpec((1, 16), lambda i: (0, 0)),
        ],
        out_specs=pl.BlockSpec((BN, 16), lambda i: (i, 0)),
        out_shape=jax.ShapeDtypeStruct((NPAD, 16), _f32),
    )


@functools.cache
def _make_latent():
    grid = NPAD // BN

    def body(x2_ref, x3_ref, zmw_ref, zmb_ref, zlw_ref, zlb_ref, eps_ref,
             z_ref, kl_ref, acc):
        i = pl.program_id(0)
        h = x2_ref[...] + x3_ref[...]
        mu = jnp.dot(h, zmw_ref[...], preferred_element_type=_f32) + zmb_ref[...]
        lv = jnp.dot(h, zlw_ref[...], preferred_element_type=_f32) + zlb_ref[...]
        z_ref[...] = mu + eps_ref[...] * jnp.exp(0.5 * lv)
        rows = lax.broadcasted_iota(_i32, (BN, 1), 0) + i * BN
        term = jnp.where(rows < N_NODES, lv - mu * mu - jnp.exp(lv), 0.0)

        @pl.when(i == 0)
        def _init():
            acc[0, 0] = 0.0

        acc[0, 0] += jnp.sum(term)

        @pl.when(i == grid - 1)
        def _fin():
            val = -0.5 * (1.0 + acc[0, 0] / (N_NODES * LAT))
            kl_ref[...] = jnp.zeros((1, 1), _f32) + val

    return pl.pallas_call(
        body,
        grid=(grid,),
        in_specs=[
            pl.BlockSpec((BN, LAT), lambda i: (i, 0)),
            pl.BlockSpec((BN, LAT), lambda i: (i, 0)),
            pl.BlockSpec((LAT, LAT), lambda i: (0, 0)),
            pl.BlockSpec((1, LAT), lambda i: (0, 0)),
            pl.BlockSpec((LAT, LAT), lambda i: (0, 0)),
            pl.BlockSpec((1, LAT), lambda i: (0, 0)),
            pl.BlockSpec((BN, LAT), lambda i: (i, 0)),
        ],
        out_specs=[
            pl.BlockSpec((BN, LAT), lambda i: (i, 0)),
            pl.BlockSpec((1, 1), lambda i: (0, 0)),
        ],
        out_shape=[
            jax.ShapeDtypeStruct((NPAD, LAT), _f32),
            jax.ShapeDtypeStruct((1, 1), _f32),
        ],
        scratch_shapes=[pltpu.SMEM((1, 1), _f32)],
    )


@functools.cache
def _make_classifier():
    bn = 400
    grid = N_NODES // bn

    def body(z_ref, w_ref, b_ref, out_ref):
        out_ref[...] = (
            jnp.dot(z_ref[...], w_ref[...], preferred_element_type=_f32)
            + b_ref[...])

    return pl.pallas_call(
        body,
        grid=(grid,),
        in_specs=[
            pl.BlockSpec((bn, LAT), lambda i: (i, 0)),
            pl.BlockSpec((LAT, NUM_IDS), lambda i: (0, 0)),
            pl.BlockSpec((1, NUM_IDS), lambda i: (0, 0)),
        ],
        out_specs=pl.BlockSpec((bn, NUM_IDS), lambda i: (i, 0)),
        out_shape=jax.ShapeDtypeStruct((N_NODES, NUM_IDS), _f32),
    )


@functools.cache
def _make_edge_mlp():
    grid = EE // BE

    def body(zs_ref, zd_ref, w1_ref, b1_ref, g1_ref, bb1_ref,
             w2_ref, b2_ref, g2_ref, bb2_ref, w3_ref, b3_ref, out_ref):
        zs = zs_ref[...]
        zd = zd_ref[...]
        ef = jnp.concatenate([zs, zd, zs * zd, jnp.abs(zs - zd)], axis=1)
        e1 = jnp.dot(ef, w1_ref[...], preferred_element_type=_f32) + b1_ref[...]
        e1 = jnp.maximum(e1 * g1_ref[...] + bb1_ref[...], 0.0)
        e2 = jnp.dot(e1, w2_ref[...], preferred_element_type=_f32) + b2_ref[...]
        e2 = jnp.maximum(e2 * g2_ref[...] + bb2_ref[...], 0.0)
        o = jnp.dot(e2, w3_ref[...], preferred_element_type=_f32) + b3_ref[...]
        out_ref[...] = jax.nn.sigmoid(o)

    return pl.pallas_call(
        body,
        grid=(grid,),
        in_specs=[
            pl.BlockSpec((BE, LAT), lambda i: (i, 0)),
            pl.BlockSpec((BE, LAT), lambda i: (i, 0)),
            pl.BlockSpec((4 * LAT, 128), lambda i: (0, 0)),
            pl.BlockSpec((1, 128), lambda i: (0, 0)),
            pl.BlockSpec((1, 128), lambda i: (0, 0)),
            pl.BlockSpec((1, 128), lambda i: (0, 0)),
            pl.BlockSpec((128, 64), lambda i: (0, 0)),
            pl.BlockSpec((1, 64), lambda i: (0, 0)),
            pl.BlockSpec((1, 64), lambda i: (0, 0)),
            pl.BlockSpec((1, 64), lambda i: (0, 0)),
            pl.BlockSpec((64, 8), lambda i: (0, 0)),
            pl.BlockSpec((1, 8), lambda i: (0, 0)),
        ],
        out_specs=pl.BlockSpec((BE, 8), lambda i: (i, 0)),
        out_shape=jax.ShapeDtypeStruct((EE, 8), _f32),
    )


_BN_SCALE = 1.0 / (1.0 + 1e-5) ** 0.5


def _row(v):
    return v.reshape(1, -1)


def _pad_rows(a, rows, val=0.0):
    return jnp.pad(a, ((0, rows - a.shape[0]), (0, 0)), constant_values=val)


def _gat_layer(xp, srcp, dstp, W, asrc, adst, bias, heads, zeros32, zeros16,
               g=None, b=None, cstore=None):
    dinp = xp.shape[1]
    if cstore is None:
        cstore = heads * 32
    hdp = cstore // heads
    asrc_p = jnp.pad(asrc, ((0, 0), (0, hdp - asrc.shape[1])))
    adst_p = jnp.pad(adst, ((0, 0), (0, hdp - adst.shape[1])))
    Wp = jnp.pad(W, ((0, dinp - W.shape[0]), (0, cstore - W.shape[1])))

    P, Q = _make_stage_a(dinp, cstore, heads)(xp, Wp, asrc_p, adst_p)
    Gh = _make_gather(NPAD, cstore, EG)(P, srcp)
    Qs = _make_gather(NPAD, 16, EG)(Q, srcp)
    Qd = _make_gather(NPAD, 16, EG)(Q, dstp)
    if cstore == 16:
        V = _make_stage_c16(EG)(Gh, Qs, Qd)
        acc = _make_scatter_add(16, EG)(V, dstp, zeros16).reshape(2, NPAD, 16)
        bias_p = _row(jnp.pad(bias, (0, 16 - bias.shape[0])))
        return _make_stage_e16()(acc, bias_p)
    V1, V2 = _make_stage_c(heads, EG)(Gh, Qs, Qd)
    accs = [
        _make_scatter_add(32, EG)(V1[hh], dstp, zeros32).reshape(2, NPAD, 32)
        for hh in range(heads)
    ]
    den = _make_scatter_add(16, EG)(V2, dstp, zeros16).reshape(2, NPAD, 16)
    bn_relu = g is not None
    if bn_relu:
        g_p = _row(g) * _BN_SCALE
        b_p = _row(b)
    else:
        g_p = _row(jnp.ones((cstore,), _f32))
        b_p = _row(jnp.zeros((cstore,), _f32))
    return _make_stage_e(heads, bn_relu)(*accs, den, _row(bias), g_p, b_p)


@jax.jit
def kernel(x, edge_index, params):
    p = params

    ids = x[:, 0].astype(_i32)
    ids_p = jnp.pad(ids, (0, IDPAD - N_NODES)).reshape(-1, K)
    emb_p = jnp.pad(p['emb'], ((0, 0), (0, 16 - EMB)))
    id_emb = _make_gather(NUM_IDS, 16, IDPAD)(emb_p, ids_p)[:NPAD, :EMB]
    feats = _pad_rows(x[:, 1:], NPAD)
    xin = jnp.concatenate(
        [id_emb, feats, jnp.zeros((NPAD, 32 - EMB - (IN_CH - 1)), _f32)], axis=1)

    loops = jnp.arange(N_NODES, dtype=_i32)
    srcg = jnp.pad(jnp.concatenate([edge_index[0], loops]),
                   (0, EG - N_EDGES - N_NODES),
                   constant_values=N_NODES).reshape(-1, K)
    dstg = jnp.pad(jnp.concatenate([edge_index[1], loops]),
                   (0, EG - N_EDGES - N_NODES),
                   constant_values=N_NODES).reshape(-1, K)
    srce = jnp.pad(edge_index[0], (0, EE - N_EDGES),
                   constant_values=N_NODES).reshape(-1, K)
    dste = jnp.pad(edge_index[1], (0, EE - N_EDGES),
                   constant_values=N_NODES).reshape(-1, K)
    zeros32 = jnp.zeros((NPAD, 32), _f32)
    zeros16 = jnp.zeros((NPAD, 16), _f32)
    eps = _pad_rows(
        jax.random.normal(jax.random.key(42), (N_NODES, LAT), dtype=_f32), NPAD)

    x1 = _gat_layer(xin, srcg, dstg, p['enc1_W'], p['enc1_asrc'],
                    p['enc1_adst'], p['enc1_b'], HEADS, zeros32, zeros16,
                    g=p['bn1_g'], b=p['bn1_b'])
    x2 = _gat_layer(x1, srcg, dstg, p['enc2_W'], p['enc2_asrc'],
                    p['enc2_adst'], p['enc2_b'], 1, zeros32, zeros16,
                    g=p['bn2_g'], b=p['bn2_b'])
    x3 = _gat_layer(x2, srcg, dstg, p['enc3_W'], p['enc3_asrc'],
                    p['enc3_adst'], p['enc3_b'], 1, zeros32, zeros16,
                    g=p['bn3_g'], b=p['bn3_b'])

    z, kl = _make_latent()(x2, x3, p['zm_W'], _row(p['zm_b']),
                           p['zl_W'], _row(p['zl_b']), eps)

    d1 = _gat_layer(z, srcg, dstg, p['dec1_W'], p['dec1_asrc'],
                    p['dec1_adst'], p['dec1_b'], HEADS, zeros32, zeros16,
                    g=p['dbn1_g'], b=p['dbn1_b'])
    d2 = _gat_layer(d1, srcg, dstg, p['dec2_W'], p['dec2_asrc'],
                    p['dec2_adst'], p['dec2_b'], 1, zeros32, zeros16,
                    g=p['dbn2_g'], b=p['dbn2_b'])
    nr = _gat_layer(d2, srcg, dstg, p['dec3_W'], p['dec3_asrc'],
                    p['dec3_adst'], p['dec3_b'], 1, zeros32, zeros16,
                    cstore=16)
    node_recon = nr[:N_NODES, :IN_CH - 1]

    canid_logits = _make_classifier()(z, p['cls_W'], _row(p['cls_b']))

    zs = _make_gather(NPAD, 32, EE)(z, srce)
    zd = _make_gather(NPAD, 32, EE)(z, dste)
    w3 = jnp.pad(p['ed3_W'], ((0, 0), (0, 7)))
    b3 = _row(jnp.pad(p['ed3_b'], (0, 7)))
    ep = _make_edge_mlp()(
        zs, zd,
        p['ed1_W'], _row(p['ed1_b']),
        _row(p['edbn1_g']) * _BN_SCALE, _row(p['edbn1_b']),
        p['ed2_W'], _row(p['ed2_b']),
        _row(p['edbn2_g']) * _BN_SCALE, _row(p['edbn2_b']),
        w3, b3)
    edge_prob = ep[:N_EDGES, :1]

    return (node_recon, canid_logits, edge_prob, kl[0, 0])
